# Initial kernel scaffold; baseline (speedup 1.0000x reference)
import jax
import jax.numpy as jnp
from jax.experimental import pallas as pl


def kernel(encoder_output, mask_embedding, unmasked_positions, gamma, beta, mask_id, unmask_id):
    raise NotImplementedError("write your pallas kernel here")


# speedup vs baseline: 4.8320x; 4.8320x over previous
import functools

import jax
import jax.numpy as jnp
from jax import lax
from jax.experimental import pallas as pl
from jax.experimental.pallas import tpu as pltpu
from jax.experimental.pallas import tpu_sc as plsc

B, T, K = 64, 1024, 768
N_MASK, N_UNMASK = 768, 256
M_ROWS = B * N_MASK
U_ROWS = B * N_UNMASK
OUT_ROWS = B * T

NC, NS = 2, 16
NW = NC * NS
CHUNK = 64
M_CHUNKS = M_ROWS // CHUNK
U_CHUNKS = U_ROWS // CHUNK
M_PER_W = M_CHUNKS // NW
U_PER_W = U_CHUNKS // NW

LN_BLK = 1024


def _ln_body(x_ref, p_ref, g_ref, b_ref, o_ref):
    x = x_ref[...] + p_ref[...]
    mu = jnp.mean(x, axis=-1, keepdims=True)
    xc = x - mu
    var = jnp.mean(xc * xc, axis=-1, keepdims=True)
    o_ref[...] = (xc / jnp.sqrt(var + 1e-5)) * g_ref[...] + b_ref[...]


def _layer_norm_tc(x, pos, gamma, beta):
    return pl.pallas_call(
        _ln_body,
        grid=(U_ROWS // LN_BLK,),
        in_specs=[
            pl.BlockSpec((LN_BLK, K), lambda i: (i, 0)),
            pl.BlockSpec((LN_BLK, K), lambda i: (i, 0)),
            pl.BlockSpec((1, K), lambda i: (0, 0)),
            pl.BlockSpec((1, K), lambda i: (0, 0)),
        ],
        out_specs=pl.BlockSpec((LN_BLK, K), lambda i: (i, 0)),
        out_shape=jax.ShapeDtypeStruct((U_ROWS, K), jnp.float32),
    )(x, pos, gamma.reshape(1, K), beta.reshape(1, K))


@functools.partial(
    pl.kernel,
    mesh=plsc.VectorSubcoreMesh(core_axis_name="c", subcore_axis_name="s"),
    out_type=jax.ShapeDtypeStruct((OUT_ROWS, K), jnp.float32),
    scratch_types=[
        pltpu.VMEM((CHUNK,), jnp.int32),
        pltpu.VMEM((CHUNK, K), jnp.float32),
        pltpu.SemaphoreType.DMA,
    ],
)
def _sc_scatter(mask_hbm, enc_hbm, midx_hbm, uidx_hbm, out_hbm,
                idx_v, rows_v, sem):
    wid = lax.axis_index("s") * NC + lax.axis_index("c")

    def do_chunk(src_hbm, idx2_hbm, chunk_id):
        pltpu.sync_copy(idx2_hbm.at[chunk_id], idx_v)
        pltpu.sync_copy(src_hbm.at[pl.ds(chunk_id * CHUNK, CHUNK)], rows_v)
        pltpu.async_copy(rows_v, out_hbm.at[idx_v], sem).wait()

    def mbody(i, carry):
        do_chunk(mask_hbm, midx_hbm, wid * M_PER_W + i)
        return carry

    lax.fori_loop(0, M_PER_W, mbody, 0)

    def ubody(i, carry):
        do_chunk(enc_hbm, uidx_hbm, wid * U_PER_W + i)
        return carry

    lax.fori_loop(0, U_PER_W, ubody, 0)


def kernel(encoder_output, mask_embedding, unmasked_positions, gamma, beta,
           mask_id, unmask_id):
    enc = _layer_norm_tc(
        encoder_output.reshape(U_ROWS, K),
        unmasked_positions.reshape(U_ROWS, K),
        gamma, beta,
    )
    bofs = (jnp.arange(B, dtype=jnp.int32) * T)[:, None]
    midx = (mask_id.astype(jnp.int32) + bofs).reshape(M_CHUNKS, CHUNK)
    uidx = (unmask_id.astype(jnp.int32) + bofs).reshape(U_CHUNKS, CHUNK)
    dec = _sc_scatter(mask_embedding.reshape(M_ROWS, K), enc, midx, uidx)
    return dec.reshape(B, T, K)


# speedup vs baseline: 5.3193x; 1.1009x over previous
import functools

import jax
import jax.numpy as jnp
from jax import lax
from jax.experimental import pallas as pl
from jax.experimental.pallas import tpu as pltpu
from jax.experimental.pallas import tpu_sc as plsc

B, T, K = 64, 1024, 768
N_MASK, N_UNMASK = 768, 256
M_ROWS = B * N_MASK
U_ROWS = B * N_UNMASK
OUT_ROWS = B * T

NC, NS = 2, 16
NW = NC * NS
CHUNK = 64
M_CHUNKS = M_ROWS // CHUNK
U_CHUNKS = U_ROWS // CHUNK
M_PER_W = M_CHUNKS // NW
U_PER_W = U_CHUNKS // NW
J_TOTAL = M_PER_W + U_PER_W

LN_BLK = 1024


def _ln_body(x_ref, p_ref, g_ref, b_ref, o_ref):
    x = x_ref[...] + p_ref[...]
    mu = jnp.mean(x, axis=-1, keepdims=True)
    xc = x - mu
    var = jnp.mean(xc * xc, axis=-1, keepdims=True)
    o_ref[...] = (xc / jnp.sqrt(var + 1e-5)) * g_ref[...] + b_ref[...]


def _layer_norm_tc(x, pos, gamma, beta):
    return pl.pallas_call(
        _ln_body,
        grid=(U_ROWS // LN_BLK,),
        in_specs=[
            pl.BlockSpec((LN_BLK, K), lambda i: (i, 0)),
            pl.BlockSpec((LN_BLK, K), lambda i: (i, 0)),
            pl.BlockSpec((1, K), lambda i: (0, 0)),
            pl.BlockSpec((1, K), lambda i: (0, 0)),
        ],
        out_specs=pl.BlockSpec((LN_BLK, K), lambda i: (i, 0)),
        out_shape=jax.ShapeDtypeStruct((U_ROWS, K), jnp.float32),
    )(x, pos, gamma.reshape(1, K), beta.reshape(1, K))


@functools.partial(
    pl.kernel,
    mesh=plsc.VectorSubcoreMesh(core_axis_name="c", subcore_axis_name="s"),
    out_type=jax.ShapeDtypeStruct((OUT_ROWS, K), jnp.float32),
    scratch_types=[
        pltpu.VMEM((J_TOTAL, CHUNK), jnp.int32),
        pltpu.VMEM((CHUNK, K), jnp.float32),
        pltpu.VMEM((CHUNK, K), jnp.float32),
        pltpu.SemaphoreType.DMA,
        pltpu.SemaphoreType.DMA,
        pltpu.SemaphoreType.DMA,
        pltpu.SemaphoreType.DMA,
    ],
)
def _sc_scatter(mask_hbm, enc_hbm, midx_hbm, uidx_hbm, out_hbm,
                idx_all, rows0, rows1, l0, l1, s0, s1):
    wid = lax.axis_index("s") * NC + lax.axis_index("c")

    pltpu.sync_copy(midx_hbm.at[pl.ds(wid * M_PER_W, M_PER_W)],
                    idx_all.at[pl.ds(0, M_PER_W)])
    pltpu.sync_copy(uidx_hbm.at[pl.ds(wid * U_PER_W, U_PER_W)],
                    idx_all.at[pl.ds(M_PER_W, U_PER_W)])

    def run_stream(src_hbm, base_chunk, jlo, npairs):
        def load(c, rows, sem):
            pltpu.async_copy(src_hbm.at[pl.ds(c * CHUNK, CHUNK)], rows, sem)

        def wait_load(rows, sem):
            pltpu.make_async_copy(
                src_hbm.at[pl.ds(0, CHUNK)], rows, sem).wait()

        def scatter(j, rows, sem):
            pltpu.async_copy(rows, out_hbm.at[idx_all.at[j]], sem)

        def wait_scatter(j, rows, sem):
            pltpu.make_async_copy(rows, out_hbm.at[idx_all.at[j]], sem).wait()

        load(base_chunk, rows0, l0)

        def pair(p, carry):
            c0 = base_chunk + 2 * p
            j0 = jlo + 2 * p
            wait_load(rows0, l0)
            scatter(j0, rows0, s0)

            @pl.when(p > 0)
            def _():
                wait_scatter(j0 - 1, rows1, s1)

            load(c0 + 1, rows1, l1)
            wait_load(rows1, l1)
            scatter(j0 + 1, rows1, s1)
            wait_scatter(j0, rows0, s0)

            @pl.when(p + 1 < npairs)
            def _():
                load(c0 + 2, rows0, l0)

            return carry

        lax.fori_loop(0, npairs, pair, 0)
        wait_scatter(jlo + 2 * npairs - 1, rows1, s1)

    run_stream(mask_hbm, wid * M_PER_W, 0, M_PER_W // 2)
    run_stream(enc_hbm, wid * U_PER_W, M_PER_W, U_PER_W // 2)


def kernel(encoder_output, mask_embedding, unmasked_positions, gamma, beta,
           mask_id, unmask_id):
    enc = _layer_norm_tc(
        encoder_output.reshape(U_ROWS, K),
        unmasked_positions.reshape(U_ROWS, K),
        gamma, beta,
    )
    bofs = (jnp.arange(B, dtype=jnp.int32) * T)[:, None]
    midx = (mask_id.astype(jnp.int32) + bofs).reshape(M_CHUNKS, CHUNK)
    uidx = (unmask_id.astype(jnp.int32) + bofs).reshape(U_CHUNKS, CHUNK)
    dec = _sc_scatter(mask_embedding.reshape(M_ROWS, K), enc, midx, uidx)
    return dec.reshape(B, T, K)


# speedup vs baseline: 5.4722x; 1.0287x over previous
import functools

import jax
import jax.numpy as jnp
from jax import lax
from jax.experimental import pallas as pl
from jax.experimental.pallas import tpu as pltpu
from jax.experimental.pallas import tpu_sc as plsc

B, T, K = 64, 1024, 768
N_MASK, N_UNMASK = 768, 256
M_ROWS = B * N_MASK
U_ROWS = B * N_UNMASK
OUT_ROWS = B * T

NC, NS = 2, 16
NW = NC * NS
CHUNK = 64
M_CHUNKS = M_ROWS // CHUNK
U_CHUNKS = U_ROWS // CHUNK
M_PER_W = M_CHUNKS // NW
U_PER_W = U_CHUNKS // NW

LN_BLK = 1024


def _ln_body(x_ref, p_ref, g_ref, b_ref, o_ref):
    x = x_ref[...] + p_ref[...]
    mu = jnp.mean(x, axis=-1, keepdims=True)
    xc = x - mu
    var = jnp.mean(xc * xc, axis=-1, keepdims=True)
    o_ref[...] = (xc / jnp.sqrt(var + 1e-5)) * g_ref[...] + b_ref[...]


def _layer_norm_tc(x, pos, gamma, beta):
    return pl.pallas_call(
        _ln_body,
        grid=(U_ROWS // LN_BLK,),
        in_specs=[
            pl.BlockSpec((LN_BLK, K), lambda i: (i, 0)),
            pl.BlockSpec((LN_BLK, K), lambda i: (i, 0)),
            pl.BlockSpec((1, K), lambda i: (0, 0)),
            pl.BlockSpec((1, K), lambda i: (0, 0)),
        ],
        out_specs=pl.BlockSpec((LN_BLK, K), lambda i: (i, 0)),
        out_shape=jax.ShapeDtypeStruct((U_ROWS, K), jnp.float32),
    )(x, pos, gamma.reshape(1, K), beta.reshape(1, K))


def _run_stream(src_hbm, out_hbm, idx_all, rows0, rows1, l0, l1, s0, s1,
                base_chunk, npairs):
    def load(c, rows, sem):
        pltpu.async_copy(src_hbm.at[pl.ds(c * CHUNK, CHUNK)], rows, sem)

    def wait_load(rows, sem):
        pltpu.make_async_copy(src_hbm.at[pl.ds(0, CHUNK)], rows, sem).wait()

    def scatter(j, rows, sem):
        pltpu.async_copy(rows, out_hbm.at[idx_all.at[j]], sem)

    def wait_scatter(j, rows, sem):
        pltpu.make_async_copy(rows, out_hbm.at[idx_all.at[j]], sem).wait()

    load(base_chunk, rows0, l0)

    def pair(p, carry):
        c0 = base_chunk + 2 * p
        j0 = 2 * p
        wait_load(rows0, l0)
        scatter(j0, rows0, s0)

        @pl.when(p > 0)
        def _():
            wait_scatter(j0 - 1, rows1, s1)

        load(c0 + 1, rows1, l1)
        wait_load(rows1, l1)
        scatter(j0 + 1, rows1, s1)
        wait_scatter(j0, rows0, s0)

        @pl.when(p + 1 < npairs)
        def _():
            load(c0 + 2, rows0, l0)

        return carry

    lax.fori_loop(0, npairs, pair, 0)
    wait_scatter(2 * npairs - 1, rows1, s1)


_SC_MESH = plsc.VectorSubcoreMesh(core_axis_name="c", subcore_axis_name="s")


@functools.partial(
    pl.kernel,
    mesh=_SC_MESH,
    out_type=jax.ShapeDtypeStruct((OUT_ROWS, K), jnp.float32),
    scratch_types=[
        pltpu.VMEM((M_PER_W, CHUNK), jnp.int32),
        pltpu.VMEM((CHUNK, K), jnp.float32),
        pltpu.VMEM((CHUNK, K), jnp.float32),
        pltpu.SemaphoreType.DMA,
        pltpu.SemaphoreType.DMA,
        pltpu.SemaphoreType.DMA,
        pltpu.SemaphoreType.DMA,
    ],
)
def _sc_scatter_mask(mask_hbm, midx_hbm, out_hbm,
                     idx_all, rows0, rows1, l0, l1, s0, s1):
    wid = lax.axis_index("s") * NC + lax.axis_index("c")
    pltpu.sync_copy(midx_hbm.at[pl.ds(wid * M_PER_W, M_PER_W)], idx_all)
    _run_stream(mask_hbm, out_hbm, idx_all, rows0, rows1, l0, l1, s0, s1,
                wid * M_PER_W, M_PER_W // 2)


@functools.partial(
    pl.kernel,
    mesh=_SC_MESH,
    out_type=(),
    scratch_types=[
        pltpu.VMEM((U_PER_W, CHUNK), jnp.int32),
        pltpu.VMEM((CHUNK, K), jnp.float32),
        pltpu.VMEM((CHUNK, K), jnp.float32),
        pltpu.SemaphoreType.DMA,
        pltpu.SemaphoreType.DMA,
        pltpu.SemaphoreType.DMA,
        pltpu.SemaphoreType.DMA,
    ],
)
def _sc_scatter_unmask(enc_hbm, uidx_hbm, dec_hbm,
                       idx_all, rows0, rows1, l0, l1, s0, s1):
    wid = lax.axis_index("s") * NC + lax.axis_index("c")
    pltpu.sync_copy(uidx_hbm.at[pl.ds(wid * U_PER_W, U_PER_W)], idx_all)
    _run_stream(enc_hbm, dec_hbm, idx_all, rows0, rows1, l0, l1, s0, s1,
                wid * U_PER_W, U_PER_W // 2)


def kernel(encoder_output, mask_embedding, unmasked_positions, gamma, beta,
           mask_id, unmask_id):
    bofs = (jnp.arange(B, dtype=jnp.int32) * T)[:, None]
    midx = (mask_id.astype(jnp.int32) + bofs).reshape(M_CHUNKS, CHUNK)
    uidx = (unmask_id.astype(jnp.int32) + bofs).reshape(U_CHUNKS, CHUNK)

    dec = _sc_scatter_mask(mask_embedding.reshape(M_ROWS, K), midx)
    enc = _layer_norm_tc(
        encoder_output.reshape(U_ROWS, K),
        unmasked_positions.reshape(U_ROWS, K),
        gamma, beta,
    )
    dec_ref = jax.new_ref(dec)
    _sc_scatter_unmask(enc, uidx, dec_ref)
    return dec_ref[...].reshape(B, T, K)
